# baseline (speedup 1.0000x reference)
import functools

import jax
import jax.numpy as jnp
from jax import lax
from jax.experimental import pallas as pl
from jax.experimental.pallas import tpu as pltpu
from jax.experimental.pallas import tpu_sc as plsc


def _make_lookup(seq: int, batch: int, vocab2: int, d: int, bw: int):
    mesh = plsc.VectorSubcoreMesh(core_axis_name="c", subcore_axis_name="s")
    n_lane = 16
    n_q = bw // n_lane

    @functools.partial(
        pl.kernel,
        mesh=mesh,
        out_type=jax.ShapeDtypeStruct((seq, d, batch), jnp.float32),
        scratch_types=[
            pltpu.VMEM((seq, bw), jnp.int32),
            pltpu.VMEM((seq, bw), jnp.int32),
            pltpu.VMEM((2, bw, 128), jnp.float32),
            pltpu.VMEM((2, d, bw), jnp.float32),
            pltpu.SemaphoreType.DMA,
            pltpu.SemaphoreType.DMA,
            pltpu.SemaphoreType.DMA,
        ],
        compiler_params=pltpu.CompilerParams(
            use_tc_tiling_on_sc=True, needs_layout_passes=False),
    )
    def lookup(w2_hbm, idx_hbm, out_hbm, idx_v, idxh_v, bufs, obufs,
               isem, gsem, osem):
        wid = lax.axis_index("s") * 2 + lax.axis_index("c")
        b0 = wid * bw

        pltpu.async_copy(idx_hbm.at[:, pl.ds(b0, bw)], idx_v, isem).wait()

        def precompute(s, carry):
            for q in range(n_q):
                v = idx_v[s, pl.ds(q * n_lane, n_lane)]
                idxh_v[s, pl.ds(q * n_lane, n_lane)] = v >> 1
            return carry

        lax.fori_loop(0, seq, precompute, 0)

        def transpose_select(s, buf, obuf):
            for q in range(n_q):
                kv = lax.iota(jnp.int32, n_lane) + q * n_lane
                off = (idx_v[s, pl.ds(q * n_lane, n_lane)] & 1) << 6
                for j in range(d):
                    vals = plsc.load_gather(buf, [kv, off + j])
                    obuf[j, pl.ds(q * n_lane, n_lane)] = vals

        def fire_gather(s, buf):
            return pltpu.async_copy(w2_hbm.at[idxh_v.at[s]], buf, gsem)

        def fire_out(s, obuf):
            dst = out_hbm.at[s, :, pl.ds(b0, bw)]
            return pltpu.async_copy(obuf, dst, osem)

        fire_gather(0, bufs.at[0])

        def body2(tt, carry):
            t0 = tt * 2
            for r in range(2):
                t = t0 + r
                buf, obuf = bufs.at[r], obufs.at[r]
                pltpu.make_async_copy(w2_hbm.at[idxh_v.at[t]], buf, gsem).wait()

                @pl.when(t + 1 < seq)
                def _():
                    fire_gather(t + 1, bufs.at[1 - r])

                @pl.when(t >= 2)
                def _():
                    pltpu.make_async_copy(
                        obuf, out_hbm.at[t - 2, :, pl.ds(b0, bw)], osem).wait()

                transpose_select(t, buf, obuf)
                fire_out(t, obuf)
            return carry

        lax.fori_loop(0, seq // 2, body2, 0)
        pltpu.make_async_copy(
            obufs.at[0], out_hbm.at[seq - 2, :, pl.ds(b0, bw)], osem).wait()
        pltpu.make_async_copy(
            obufs.at[1], out_hbm.at[seq - 1, :, pl.ds(b0, bw)], osem).wait()

    return lookup


def kernel(trg_seq, enc_output, W):
    del enc_output
    batch, seq = trg_seq.shape
    v, d = W.shape

    n_workers = 32
    bw = batch // n_workers

    w2 = W.reshape(v // 2, 2 * d)
    idx_t = trg_seq.T.astype(jnp.int32)

    fn = _make_lookup(seq, batch, v // 2, d, bw)
    out_t = fn(w2, idx_t)
    return jnp.transpose(out_t, (2, 0, 1))


# speedup vs baseline: 1.8355x; 1.8355x over previous
import functools

import jax
import jax.numpy as jnp
from jax import lax
from jax.experimental import pallas as pl
from jax.experimental.pallas import tpu as pltpu
from jax.experimental.pallas import tpu_sc as plsc

_NBUF = 2


def _make_gather(n_workers: int, per_w: int, chunk: int, n_ch: int,
                 n_total: int, d: int):
    mesh = plsc.VectorSubcoreMesh(core_axis_name="c", subcore_axis_name="s")

    @functools.partial(
        pl.kernel,
        mesh=mesh,
        out_type=jax.ShapeDtypeStruct((n_total, d), jnp.float32),
        scratch_types=[
            pltpu.VMEM((n_ch, chunk), jnp.int32),
            pltpu.VMEM((_NBUF, chunk, 2 * d), jnp.float32),
            pltpu.VMEM((_NBUF, chunk, d), jnp.float32),
            pltpu.SemaphoreType.DMA,
            pltpu.SemaphoreType.DMA,
        ],
        compiler_params=pltpu.CompilerParams(
            use_tc_tiling_on_sc=True, needs_layout_passes=False),
    )
    def gather_kernel(table_hbm, idx_hbm, out_hbm, idx_v, bufs, obufs,
                      gsem, osem):
        wid = lax.axis_index("s") * 2 + lax.axis_index("c")
        base = wid * per_w
        pltpu.sync_copy(idx_hbm.at[wid], idx_v)

        n_outer = n_ch // _NBUF
        n_lane = 16

        def compact(buf, obuf):
            for k in range(chunk):
                for q in range(d // n_lane):
                    obuf[k, pl.ds(q * n_lane, n_lane)] = (
                        buf[k, pl.ds(q * n_lane, n_lane)])

        def body(jj, carry):
            j0 = jj * _NBUF
            gh = [
                pltpu.async_copy(table_hbm.at[idx_v.at[j0 + b]],
                                 bufs.at[b], gsem)
                for b in range(_NBUF)
            ]
            oh = []
            for b in range(_NBUF):
                gh[b].wait()
                compact(bufs.at[b], obufs.at[b])
                dst = out_hbm.at[pl.ds(base + (j0 + b) * chunk, chunk)]
                oh.append(pltpu.async_copy(obufs.at[b], dst, osem))
            for b in range(_NBUF):
                oh[b].wait()
            return carry

        lax.fori_loop(0, n_outer, body, 0)

    return gather_kernel


def kernel(trg_seq, enc_output, W):
    del enc_output
    batch, seq = trg_seq.shape
    v, d = W.shape
    n_total = batch * seq

    n_workers = 32
    per_w = n_total // n_workers
    chunk = 128
    n_ch = per_w // chunk

    w128 = jnp.pad(W, ((0, 0), (0, d)))
    idx = trg_seq.reshape(n_workers, n_ch, chunk).astype(jnp.int32)
    fn = _make_gather(n_workers, per_w, chunk, n_ch, n_total, d)
    out = fn(w128, idx)
    return out.reshape(batch, seq, d)


# speedup vs baseline: 1.9228x; 1.0475x over previous
import functools

import jax
import jax.numpy as jnp
from jax import lax
from jax.experimental import pallas as pl
from jax.experimental.pallas import tpu as pltpu
from jax.experimental.pallas import tpu_sc as plsc

_NBUF = 2


def _make_gather(n_workers: int, per_w: int, chunk: int, n_ch: int,
                 n_total: int, d: int):
    mesh = plsc.VectorSubcoreMesh(core_axis_name="c", subcore_axis_name="s")

    @functools.partial(
        pl.kernel,
        mesh=mesh,
        out_type=jax.ShapeDtypeStruct((n_total, d), jnp.float32),
        scratch_types=[
            pltpu.VMEM((n_ch, chunk), jnp.int32),
            pltpu.VMEM((_NBUF, chunk, 2 * d), jnp.float32),
            pltpu.VMEM((_NBUF, chunk, d), jnp.float32),
            pltpu.SemaphoreType.DMA,
            pltpu.SemaphoreType.DMA,
        ],
        compiler_params=pltpu.CompilerParams(
            use_tc_tiling_on_sc=True, needs_layout_passes=False),
    )
    def gather_kernel(table_hbm, idx_hbm, out_hbm, idx_v, bufs, obufs,
                      gsem, osem):
        wid = lax.axis_index("s") * 2 + lax.axis_index("c")
        base = wid * per_w
        pltpu.sync_copy(idx_hbm.at[wid], idx_v)

        n_outer = n_ch // _NBUF
        n_lane = 16

        def compact(buf, obuf):
            for k in range(chunk):
                for q in range(d // n_lane):
                    obuf[k, pl.ds(q * n_lane, n_lane)] = (
                        buf[k, pl.ds(q * n_lane, n_lane)])

        def fire_gather(j, b):
            return pltpu.async_copy(table_hbm.at[idx_v.at[j]],
                                    bufs.at[b], gsem)

        def wait_out(j, b):
            pltpu.make_async_copy(
                obufs.at[b],
                out_hbm.at[pl.ds(base + j * chunk, chunk)], osem).wait()

        for b in range(_NBUF):
            fire_gather(b, b)

        def body(jj, carry):
            j0 = jj * _NBUF
            for b in range(_NBUF):
                j = j0 + b
                pltpu.make_async_copy(table_hbm.at[idx_v.at[j]],
                                      bufs.at[b], gsem).wait()

                @pl.when(j >= _NBUF)
                def _():
                    wait_out(j - _NBUF, b)

                compact(bufs.at[b], obufs.at[b])

                @pl.when(j + _NBUF < n_ch)
                def _():
                    fire_gather(j + _NBUF, b)

                dst = out_hbm.at[pl.ds(base + j * chunk, chunk)]
                pltpu.async_copy(obufs.at[b], dst, osem)
            return carry

        lax.fori_loop(0, n_outer, body, 0)
        for b in range(_NBUF):
            wait_out(n_ch - _NBUF + b, b)

    return gather_kernel


def kernel(trg_seq, enc_output, W):
    del enc_output
    batch, seq = trg_seq.shape
    v, d = W.shape
    n_total = batch * seq

    n_workers = 32
    per_w = n_total // n_workers
    chunk = 128
    n_ch = per_w // chunk

    w128 = jnp.pad(W, ((0, 0), (0, d)))
    idx = trg_seq.reshape(n_workers, n_ch, chunk).astype(jnp.int32)
    fn = _make_gather(n_workers, per_w, chunk, n_ch, n_total, d)
    out = fn(w128, idx)
    return out.reshape(batch, seq, d)
